# baseline (speedup 1.0000x reference)
import functools

import jax
import jax.numpy as jnp
from jax import lax
from jax.experimental import pallas as pl
from jax.experimental.pallas import tpu as pltpu
from jax.experimental.pallas import tpu_sc as plsc

_NC = 2
_NS = 16
_NW = _NC * _NS

_D = 64
_SUB = 128
_NSUB = 5
_CH = _SUB * _NSUB


def _gather(weight, idx2d):
    n = idx2d.shape[0] * _SUB
    b_per_w = n // _NW
    nchunks = b_per_w // _CH
    rows_per_w = b_per_w // _SUB

    mesh = plsc.VectorSubcoreMesh(core_axis_name="c", subcore_axis_name="s")

    @functools.partial(
        pl.kernel,
        out_type=jax.ShapeDtypeStruct((n, _D), jnp.float32),
        mesh=mesh,
        scratch_types=[
            pltpu.VMEM((rows_per_w, _SUB), jnp.int32),
            pltpu.VMEM((2, _CH, _D), jnp.float32),
            pltpu.SemaphoreType.DMA,
        ],
        compiler_params=pltpu.CompilerParams(use_tc_tiling_on_sc=False),
    )
    def k(w_hbm, idx_hbm, out_hbm, idx_v, rows_v, gsem):
        wid = lax.axis_index("s") * _NC + lax.axis_index("c")
        row0 = wid * rows_per_w

        pltpu.sync_copy(idx_hbm.at[pl.ds(row0, rows_per_w)], idx_v)

        def fire_gathers(b, irow):
            return [
                pltpu.async_copy(
                    w_hbm.at[idx_v.at[irow + j]],
                    rows_v.at[b, pl.ds(j * _SUB, _SUB)],
                    gsem,
                )
                for j in range(_NSUB)
            ]

        @pl.loop(0, nchunks // 2)
        def _pair(i):
            irow0 = i * 2 * _NSUB

            cps_a = fire_gathers(0, irow0)

            @pl.when(i > 0)
            def _wb_prev():
                pltpu.sync_copy(
                    rows_v.at[1],
                    out_hbm.at[pl.ds((row0 + irow0 - _NSUB) * _SUB, _CH)],
                )

            for cp in cps_a:
                cp.wait()

            cps_b = fire_gathers(1, irow0 + _NSUB)
            pltpu.sync_copy(
                rows_v.at[0], out_hbm.at[pl.ds((row0 + irow0) * _SUB, _CH)]
            )
            for cp in cps_b:
                cp.wait()

        last_irow = row0 + (nchunks - 1) * _NSUB
        pltpu.sync_copy(rows_v.at[1], out_hbm.at[pl.ds(last_irow * _SUB, _CH)])

    return k(weight, idx2d)


def kernel(x, orig_weight, label_embedding, prompt_embedding):
    weight = jnp.concatenate(
        [orig_weight, label_embedding, prompt_embedding], axis=0
    )
    batch, hist = x.shape
    half = hist // 2
    outs = []
    for xh in (x[:, :half], x[:, half:]):
        oh = _gather(weight, xh.reshape(-1, _SUB))
        outs.append(oh.reshape(batch, half, _D))
    return jnp.concatenate(outs, axis=1)


# speedup vs baseline: 1.1428x; 1.1428x over previous
import functools

import jax
import jax.numpy as jnp
from jax import lax
from jax.experimental import pallas as pl
from jax.experimental.pallas import tpu as pltpu
from jax.experimental.pallas import tpu_sc as plsc

_NC = 2
_NS = 16
_NW = _NC * _NS

_D = 64
_SUB = 128
_NSUB = 5
_CH = _SUB * _NSUB


def _gather(weight, idx2d):
    n = idx2d.shape[0] * _SUB
    b_per_w = n // _NW
    nchunks = b_per_w // _CH
    rows_per_w = b_per_w // _SUB

    mesh = plsc.VectorSubcoreMesh(core_axis_name="c", subcore_axis_name="s")

    @functools.partial(
        pl.kernel,
        out_type=jax.ShapeDtypeStruct((n, _D), jnp.float32),
        mesh=mesh,
        scratch_types=[
            pltpu.VMEM((rows_per_w, _SUB), jnp.int32),
            pltpu.VMEM((2, _CH, _D), jnp.float32),
            pltpu.SemaphoreType.DMA,
        ],
        compiler_params=pltpu.CompilerParams(use_tc_tiling_on_sc=False),
    )
    def k(w_hbm, idx_hbm, out_hbm, idx_v, rows_v, gsem):
        wid = lax.axis_index("s") * _NC + lax.axis_index("c")
        row0 = wid * rows_per_w

        pltpu.sync_copy(idx_hbm.at[pl.ds(row0, rows_per_w)], idx_v)

        def fire_gathers(b, irow):
            return [
                pltpu.async_copy(
                    w_hbm.at[idx_v.at[irow + j]],
                    rows_v.at[b, pl.ds(j * _SUB, _SUB)],
                    gsem,
                )
                for j in range(_NSUB)
            ]

        @pl.loop(0, nchunks // 2)
        def _pair(i):
            irow0 = i * 2 * _NSUB

            cps_a = fire_gathers(0, irow0)

            @pl.when(i > 0)
            def _wb_prev():
                pltpu.sync_copy(
                    rows_v.at[1],
                    out_hbm.at[pl.ds((row0 + irow0 - _NSUB) * _SUB, _CH)],
                )

            cps_b = fire_gathers(1, irow0 + _NSUB)
            for cp in cps_a:
                cp.wait()

            pltpu.sync_copy(
                rows_v.at[0], out_hbm.at[pl.ds((row0 + irow0) * _SUB, _CH)]
            )
            for cp in cps_b:
                cp.wait()

        last_irow = row0 + (nchunks - 1) * _NSUB
        pltpu.sync_copy(rows_v.at[1], out_hbm.at[pl.ds(last_irow * _SUB, _CH)])

    return k(weight, idx2d)


def kernel(x, orig_weight, label_embedding, prompt_embedding):
    weight = jnp.concatenate(
        [orig_weight, label_embedding, prompt_embedding], axis=0
    )
    idx2d = x.reshape(-1, _SUB)
    out = _gather(weight, idx2d)
    return out.reshape(x.shape + (orig_weight.shape[-1],))


# speedup vs baseline: 1.1782x; 1.0309x over previous
import functools

import jax
import jax.numpy as jnp
from jax import lax
from jax.experimental import pallas as pl
from jax.experimental.pallas import tpu as pltpu
from jax.experimental.pallas import tpu_sc as plsc

_NC = 2
_NS = 16
_NW = _NC * _NS

_D = 64
_SUB = 128
_NSUB = 5
_CH = _SUB * _NSUB


def _gather(weight, idx2d):
    n = idx2d.shape[0] * _SUB
    b_per_w = n // _NW
    nchunks = b_per_w // _CH
    rows_per_w = b_per_w // _SUB

    mesh = plsc.VectorSubcoreMesh(core_axis_name="c", subcore_axis_name="s")

    @functools.partial(
        pl.kernel,
        out_type=jax.ShapeDtypeStruct((n, _D), jnp.float32),
        mesh=mesh,
        scratch_types=[
            pltpu.VMEM((rows_per_w, _SUB), jnp.int32),
            pltpu.VMEM((2, _CH, _D), jnp.float32),
            pltpu.SemaphoreType.DMA,
        ],
        compiler_params=pltpu.CompilerParams(use_tc_tiling_on_sc=False),
    )
    def k(w_hbm, idx_hbm, out_hbm, idx_v, rows_v, gsem):
        wid = lax.axis_index("s") * _NC + lax.axis_index("c")
        row0 = wid * rows_per_w

        pltpu.sync_copy(idx_hbm.at[pl.ds(row0, rows_per_w)], idx_v)

        def fire_gathers(b, irow):
            return [
                pltpu.async_copy(
                    w_hbm.at[idx_v.at[irow + j]],
                    rows_v.at[b, pl.ds(j * _SUB, _SUB)],
                    gsem,
                )
                for j in range(_NSUB)
            ]

        @pl.loop(0, nchunks // 2)
        def _pair(i):
            irow0 = i * 2 * _NSUB

            cps_a = fire_gathers(0, irow0)

            @pl.when(i > 0)
            def _wb_prev():
                pltpu.sync_copy(
                    rows_v.at[1],
                    out_hbm.at[pl.ds((row0 + irow0 - _NSUB) * _SUB, _CH)],
                )

            cps_b = fire_gathers(1, irow0 + _NSUB)
            for cp in cps_a:
                cp.wait()

            pltpu.sync_copy(
                rows_v.at[0], out_hbm.at[pl.ds((row0 + irow0) * _SUB, _CH)]
            )
            for cp in cps_b:
                cp.wait()

        last_irow = row0 + (nchunks - 1) * _NSUB
        pltpu.sync_copy(rows_v.at[1], out_hbm.at[pl.ds(last_irow * _SUB, _CH)])

    return k(weight, idx2d)


def kernel(x, orig_weight, label_embedding, prompt_embedding):
    weight = jnp.concatenate(
        [orig_weight.reshape(-1), label_embedding.reshape(-1),
         prompt_embedding.reshape(-1)]
    ).reshape(-1, orig_weight.shape[-1])
    idx2d = x.reshape(-1, _SUB)
    out = _gather(weight, idx2d)
    return out.reshape(x.shape + (orig_weight.shape[-1],))
